# Initial kernel scaffold; baseline (speedup 1.0000x reference)
import jax
import jax.numpy as jnp
from jax.experimental import pallas as pl


def kernel(x, table):
    raise NotImplementedError("write your pallas kernel here")


# speedup vs baseline: 3.7650x; 3.7650x over previous
import functools

import jax
import jax.numpy as jnp
from jax import lax
from jax.experimental import pallas as pl
from jax.experimental.pallas import tpu as pltpu
from jax.experimental.pallas import tpu_sc as plsc

MAX_LEN = 8192
D_MODEL = 768
BATCH = 4
NUM_CORES = 2
NUM_SUBCORES = 16
NW = NUM_CORES * NUM_SUBCORES
POS_PER_W = MAX_LEN // NW
CHUNK = 128
NCHUNK = POS_PER_W // CHUNK

_mesh = plsc.VectorSubcoreMesh(core_axis_name="c", subcore_axis_name="s")


@functools.partial(
    pl.kernel,
    mesh=_mesh,
    out_type=jax.ShapeDtypeStruct((BATCH * MAX_LEN, D_MODEL), jnp.float32),
    scratch_types=[
        pltpu.VMEM((CHUNK,), jnp.int32),
        pltpu.VMEM((CHUNK, D_MODEL), jnp.float32),
        pltpu.SemaphoreType.DMA,
    ],
)
def _rel_pos_emb(table_hbm, out_hbm, idx_v, rows_v, sem):
    wid = lax.axis_index("s") * NUM_CORES + lax.axis_index("c")
    base = wid * POS_PER_W
    for c in range(NCHUNK):
        cbase = base + c * CHUNK
        for j in range(CHUNK // 16):
            p = cbase + j * 16 + lax.iota(jnp.int32, 16)
            idx_v[pl.ds(j * 16, 16)] = jnp.abs(p - MAX_LEN // 2)
        pltpu.async_copy(table_hbm.at[idx_v], rows_v, sem).wait()
        for b in range(BATCH):
            pltpu.sync_copy(rows_v, out_hbm.at[pl.ds(b * MAX_LEN + cbase, CHUNK)])


def kernel(x, table):
    del x
    out = _rel_pos_emb(table)
    return out.reshape(BATCH, MAX_LEN, D_MODEL)


# speedup vs baseline: 4.0029x; 1.0632x over previous
import functools

import jax
import jax.numpy as jnp
from jax import lax
from jax.experimental import pallas as pl
from jax.experimental.pallas import tpu as pltpu
from jax.experimental.pallas import tpu_sc as plsc

MAX_LEN = 8192
HALF = MAX_LEN // 2
D_MODEL = 768
BATCH = 4
NUM_CORES = 2
NUM_SUBCORES = 16
NW = NUM_CORES * NUM_SUBCORES
ROWS_PER_W = HALF // NW

_mesh = plsc.VectorSubcoreMesh(core_axis_name="c", subcore_axis_name="s")


@functools.partial(
    pl.kernel,
    mesh=_mesh,
    out_type=jax.ShapeDtypeStruct((BATCH * MAX_LEN, D_MODEL), jnp.float32),
    scratch_types=[
        pltpu.VMEM((ROWS_PER_W, D_MODEL), jnp.float32),
        pltpu.VMEM((ROWS_PER_W,), jnp.int32),
        pltpu.VMEM((ROWS_PER_W,), jnp.int32),
        pltpu.VMEM((ROWS_PER_W,), jnp.int32),
        pltpu.VMEM((ROWS_PER_W,), jnp.int32),
        pltpu.VMEM((16, D_MODEL), jnp.float32),
        pltpu.VMEM((16,), jnp.int32),
        pltpu.VMEM((16,), jnp.int32),
        pltpu.SemaphoreType.DMA,
    ],
)
def _rel_pos_emb(table_hbm, out_hbm, rows_v, i0, i1, i2, i3,
                 spec_v, gidx, oidx, sem):
    wid = lax.axis_index("s") * NUM_CORES + lax.axis_index("c")
    rbase = wid * ROWS_PER_W
    pltpu.sync_copy(table_hbm.at[pl.ds(rbase, ROWS_PER_W)], rows_v)

    ridx = [i0, i1, i2, i3]
    for b in range(BATCH):
        for t in range(ROWS_PER_W // 16):
            head = b * MAX_LEN + HALF - rbase - t * 16
            ridx[b][pl.ds(t * 16, 16)] = head - lax.iota(jnp.int32, 16)

    for b in range(BATCH):
        pltpu.sync_copy(
            rows_v,
            out_hbm.at[pl.ds(b * MAX_LEN + HALF + rbase, ROWS_PER_W)])
        pltpu.sync_copy(rows_v, out_hbm.at[ridx[b]])

    @pl.when(wid < BATCH)
    def _patch():
        gidx[...] = HALF - lax.iota(jnp.int32, 16)
        oidx[...] = wid * MAX_LEN + lax.iota(jnp.int32, 16)
        pltpu.async_copy(table_hbm.at[gidx], spec_v, sem).wait()
        pltpu.async_copy(spec_v, out_hbm.at[oidx], sem).wait()


def kernel(x, table):
    del x
    out = _rel_pos_emb(table)
    return out.reshape(BATCH, MAX_LEN, D_MODEL)


# speedup vs baseline: 4.0224x; 1.0049x over previous
import functools

import jax
import jax.numpy as jnp
from jax import lax
from jax.experimental import pallas as pl
from jax.experimental.pallas import tpu as pltpu
from jax.experimental.pallas import tpu_sc as plsc

MAX_LEN = 8192
HALF = MAX_LEN // 2
D_MODEL = 768
BATCH = 4
NUM_CORES = 2
NUM_SUBCORES = 16
NW = NUM_CORES * NUM_SUBCORES
ROWS_PER_W = HALF // NW

_mesh = plsc.VectorSubcoreMesh(core_axis_name="c", subcore_axis_name="s")


@functools.partial(
    pl.kernel,
    mesh=_mesh,
    out_type=jax.ShapeDtypeStruct((BATCH * MAX_LEN, D_MODEL), jnp.float32),
    scratch_types=[
        pltpu.VMEM((ROWS_PER_W, D_MODEL), jnp.float32),
        pltpu.VMEM((ROWS_PER_W,), jnp.int32),
        pltpu.VMEM((ROWS_PER_W,), jnp.int32),
        pltpu.VMEM((ROWS_PER_W,), jnp.int32),
        pltpu.VMEM((ROWS_PER_W,), jnp.int32),
        pltpu.VMEM((16, D_MODEL), jnp.float32),
        pltpu.VMEM((16,), jnp.int32),
        pltpu.VMEM((16,), jnp.int32),
        pltpu.SemaphoreType.DMA,
        pltpu.SemaphoreType.DMA,
    ],
)
def _rel_pos_emb(table_hbm, out_hbm, rows_v, i0, i1, i2, i3,
                 spec_v, gidx, oidx, sem, psem):
    wid = lax.axis_index("s") * NUM_CORES + lax.axis_index("c")
    rbase = wid * ROWS_PER_W
    pltpu.sync_copy(table_hbm.at[pl.ds(rbase, ROWS_PER_W)], rows_v)

    ridx = [i0, i1, i2, i3]
    for b in range(BATCH):
        for t in range(ROWS_PER_W // 16):
            head = b * MAX_LEN + HALF - rbase - t * 16
            ridx[b][pl.ds(t * 16, 16)] = head - lax.iota(jnp.int32, 16)

    copies = []
    for b in range(BATCH):
        copies.append(pltpu.async_copy(
            rows_v,
            out_hbm.at[pl.ds(b * MAX_LEN + HALF + rbase, ROWS_PER_W)],
            sem))
        copies.append(pltpu.async_copy(rows_v, out_hbm.at[ridx[b]], sem))

    @pl.when(wid < BATCH)
    def _patch():
        gidx[...] = HALF - lax.iota(jnp.int32, 16)
        oidx[...] = wid * MAX_LEN + lax.iota(jnp.int32, 16)
        pltpu.async_copy(table_hbm.at[gidx], spec_v, psem).wait()
        pltpu.async_copy(spec_v, out_hbm.at[oidx], psem).wait()

    for c in copies:
        c.wait()


def kernel(x, table):
    del x
    out = _rel_pos_emb(table)
    return out.reshape(BATCH, MAX_LEN, D_MODEL)
